# Initial kernel scaffold; baseline (speedup 1.0000x reference)
import functools

import numpy as np
import jax
import jax.numpy as jnp
from jax import lax
from jax.experimental import pallas as pl
from jax.experimental.pallas import tpu as pltpu
from jax.experimental.pallas import tpu_sc as plsc

_N_RAYS = 4096
_N_POINTS = 200
_NUM_LEVELS = 4
_RES = (32, 64, 128, 256)
_TS = (35937, 274625, 524288, 524288)
_OFF = (0, 35937, 310562, 834850)
_TPTOT = 1359138
_TPTOT_PAD = 1359360
_P1 = np.int32(np.uint32(2654435761).view(np.int32))
_P2 = np.int32(805459861)
_DENSE = (True, True, False, False)

_SEL = (np.arange(128)[:, None] // 8 == np.arange(16)[None, :]).astype(np.float32)


def _proj_body(t_ref, s_ref, o_ref):
    o_ref[...] = jnp.dot(t_ref[...], s_ref[...],
                         preferred_element_type=jnp.float32)


def _project_level(l, table, W):
    T = _TS[l]
    Tp = ((T + 15) // 16) * 16
    R = Tp // 16
    tf = jnp.pad(table, ((0, Tp - T), (0, 0))).reshape(R, 128)
    wl = W[8 * l:8 * l + 8, 0]
    S = jnp.asarray(_SEL) * jnp.tile(wl, 16)[:, None]
    BR = 1024
    grid = (R + BR - 1) // BR
    out = pl.pallas_call(
        _proj_body,
        grid=(grid,),
        in_specs=[pl.BlockSpec((BR, 128), lambda i: (i, 0)),
                  pl.BlockSpec((128, 16), lambda i: (0, 0))],
        out_specs=pl.BlockSpec((BR, 16), lambda i: (i, 0)),
        out_shape=jax.ShapeDtypeStruct((R, 16), jnp.float32),
    )(tf, S)
    return out.reshape(-1)[:T]


def _rays_body(x_ref, o_ref):
    xb = x_ref[...]
    th1, ph1, th2, ph2 = (xb[:, i:i + 1] for i in range(4))
    s1, c1 = jnp.sin(th1), jnp.cos(th1)
    s2, c2 = jnp.sin(th2), jnp.cos(th2)
    p1 = (s1 * jnp.cos(ph1), s1 * jnp.sin(ph1), c1)
    p2 = (s2 * jnp.cos(ph2), s2 * jnp.sin(ph2), c2)
    cols = [(p1[k] + 1.0) * 0.5 for k in range(3)]
    cols += [(p2[k] - p1[k]) * 0.5 for k in range(3)]
    z = jnp.zeros_like(th1)
    o_ref[...] = jnp.concatenate(cols + [z, z], axis=1)


def _rays_tc(x):
    return pl.pallas_call(
        _rays_body,
        out_shape=jax.ShapeDtypeStruct((_N_RAYS, 8), jnp.float32),
    )(x)


_mesh = plsc.VectorSubcoreMesh(core_axis_name="c", subcore_axis_name="s",
                               num_cores=2, num_subcores=16)


@functools.partial(
    pl.kernel,
    out_type=jax.ShapeDtypeStruct((_N_RAYS,), jnp.float32),
    mesh=_mesh,
    scratch_types=[
        pltpu.VMEM((128, 8), jnp.float32),
        pltpu.VMEM((16,), jnp.float32),
        pltpu.VMEM((4, 128), jnp.int32),
        pltpu.VMEM((4, 128), jnp.float32),
        pltpu.VMEM((128,), jnp.float32),
        pltpu.SemaphoreType.DMA,
    ],
)
def _sc_main(tp_hbm, rays_hbm, b_hbm, out_hbm,
             rays_v, b_v, idx_v, g_v, out_v, sem):
    wid = lax.axis_index("s") * 2 + lax.axis_index("c")
    base = pl.multiple_of(wid * 128, 128)
    pltpu.sync_copy(rays_hbm.at[pl.ds(base, 128)], rays_v)
    pltpu.sync_copy(b_hbm, b_v)
    bv = b_v[...]
    lane = jnp.arange(16, dtype=jnp.int32)

    for grp in range(8):
        rvec = grp * 16 + lane
        pa = [plsc.load_gather(rays_v, [rvec, jnp.full((16,), p, jnp.int32)])
              for p in range(6)]
        ax, ay, az, dx, dy, dz = pa

        def body(it, m):
            tf = it.astype(jnp.float32) * jnp.float32(1.0 / 199.0)
            x0 = jnp.clip(ax + dx * tf, 0.0, 1.0)
            x1 = jnp.clip(ay + dy * tf, 0.0, 1.0)
            x2 = jnp.clip(az + dz * tf, 0.0, 1.0)
            wlist = []
            for l in range(_NUM_LEVELS):
                r1 = jnp.float32(_RES[l] - 1)
                p0 = x0 * r1
                p1 = x1 * r1
                p2 = x2 * r1
                i0 = p0.astype(jnp.int32)
                i1 = p1.astype(jnp.int32)
                i2 = p2.astype(jnp.int32)
                f0 = p0 - i0.astype(jnp.float32)
                f1 = p1 - i1.astype(jnp.float32)
                f2 = p2 - i2.astype(jnp.float32)
                g0 = 1.0 - f0
                g1 = 1.0 - f1
                g2 = 1.0 - f2
                yz = (g1 * g2, f1 * g2, g1 * f2, f1 * f2)
                if _DENSE[l]:
                    s = _RES[l] + 1
                    bidx = i0 + i1 * s + i2 * (s * s) + _OFF[l]
                else:
                    y0 = i1 * _P1
                    y1 = y0 + _P1
                    z0 = i2 * _P2
                    z1 = z0 + _P2
                    msk = np.int32(_TS[l] - 1)
                for c8 in range(8):
                    ox, oy, oz = c8 & 1, (c8 >> 1) & 1, (c8 >> 2) & 1
                    w = (f0 if ox else g0) * yz[oy + 2 * oz]
                    wlist.append(w)
                    if _DENSE[l]:
                        s = _RES[l] + 1
                        idx = bidx + (ox + oy * s + oz * (s * s))
                    else:
                        tx = (i0 + 1) if ox else i0
                        ty = y1 if oy else y0
                        tz = z1 if oz else z0
                        idx = ((tx ^ ty ^ tz) & msk) + _OFF[l]
                    idx_v[l, pl.ds(c8 * 16, 16)] = idx
            cps = [pltpu.async_copy(tp_hbm.at[idx_v.at[l]], g_v.at[l], sem)
                   for l in range(_NUM_LEVELS)]
            for cp in cps:
                cp.wait()
            z = jnp.zeros((16,), jnp.float32)
            k = 0
            for l in range(_NUM_LEVELS):
                for c8 in range(8):
                    z = z + wlist[k] * g_v[l, pl.ds(c8 * 16, 16)]
                    k += 1
            return jnp.maximum(m, z)

        m = lax.fori_loop(0, _N_POINTS, body,
                          jnp.full((16,), -3e38, jnp.float32))
        out_v[pl.ds(grp * 16, 16)] = 1.0 / (1.0 + jnp.exp(-(m + bv)))

    pltpu.sync_copy(out_v, out_hbm.at[pl.ds(base, 128)])


def kernel(x, table0, table1, table2, table3, W, b):
    tabs = (table0, table1, table2, table3)
    projs = [_project_level(l, tabs[l], W) for l in range(_NUM_LEVELS)]
    projs.append(jnp.zeros((_TPTOT_PAD - _TPTOT,), jnp.float32))
    tp = jnp.concatenate(projs)
    rays = _rays_tc(x)
    bvec = jnp.broadcast_to(b.astype(jnp.float32), (16,))
    out = _sc_main(tp, rays, bvec)
    return out.reshape(_N_RAYS, 1)


# speedup vs baseline: 29.6758x; 29.6758x over previous
import functools

import numpy as np
import jax
import jax.numpy as jnp
from jax import lax
from jax.experimental import pallas as pl
from jax.experimental.pallas import tpu as pltpu
from jax.experimental.pallas import tpu_sc as plsc

_N_RAYS = 4096
_N_POINTS = 200
_NUM_LEVELS = 4
_RES = (32, 64, 128, 256)
_TS = (35937, 274625, 524288, 524288)
_OFF = (0, 35937, 310562, 834850)
_TPTOT = 1359138
_TPTOT_PAD = 1359360
_P1 = np.int32(np.uint32(2654435761).view(np.int32))
_P2 = np.int32(805459861)
_DENSE = (True, True, False, False)

_SEL = (np.arange(128)[:, None] // 8 == np.arange(16)[None, :]).astype(np.float32)


def _proj_body(t_ref, s_ref, o_ref):
    o_ref[...] = jnp.dot(t_ref[...], s_ref[...],
                         preferred_element_type=jnp.float32)


def _project_level(l, table, W):
    T = _TS[l]
    Tp = ((T + 15) // 16) * 16
    R = Tp // 16
    tf = jnp.pad(table, ((0, Tp - T), (0, 0))).reshape(R, 128)
    wl = W[8 * l:8 * l + 8, 0]
    S = jnp.asarray(_SEL) * jnp.tile(wl, 16)[:, None]
    BR = 1024
    grid = (R + BR - 1) // BR
    out = pl.pallas_call(
        _proj_body,
        grid=(grid,),
        in_specs=[pl.BlockSpec((BR, 128), lambda i: (i, 0)),
                  pl.BlockSpec((128, 16), lambda i: (0, 0))],
        out_specs=pl.BlockSpec((BR, 16), lambda i: (i, 0)),
        out_shape=jax.ShapeDtypeStruct((R, 16), jnp.float32),
    )(tf, S)
    return out.reshape(-1)[:T]


def _rays_body(x_ref, o_ref):
    xb = x_ref[...]
    th1, ph1, th2, ph2 = (xb[:, i:i + 1] for i in range(4))
    s1, c1 = jnp.sin(th1), jnp.cos(th1)
    s2, c2 = jnp.sin(th2), jnp.cos(th2)
    p1 = (s1 * jnp.cos(ph1), s1 * jnp.sin(ph1), c1)
    p2 = (s2 * jnp.cos(ph2), s2 * jnp.sin(ph2), c2)
    cols = [(p1[k] + 1.0) * 0.5 for k in range(3)]
    cols += [(p2[k] - p1[k]) * 0.5 for k in range(3)]
    z = jnp.zeros_like(th1)
    o_ref[...] = jnp.concatenate(cols + [z, z], axis=1)


def _rays_tc(x):
    return pl.pallas_call(
        _rays_body,
        out_shape=jax.ShapeDtypeStruct((_N_RAYS, 8), jnp.float32),
    )(x)


_mesh = plsc.VectorSubcoreMesh(core_axis_name="c", subcore_axis_name="s",
                               num_cores=2, num_subcores=16)


@functools.partial(
    pl.kernel,
    out_type=jax.ShapeDtypeStruct((_N_RAYS,), jnp.float32),
    mesh=_mesh,
    scratch_types=[
        pltpu.VMEM((1024,), jnp.float32),
        pltpu.VMEM((16,), jnp.float32),
        pltpu.VMEM((4, 128), jnp.int32),
        pltpu.VMEM((4, 128), jnp.float32),
        pltpu.VMEM((128,), jnp.float32),
        pltpu.SemaphoreType.DMA,
    ],
    compiler_params=pltpu.CompilerParams(needs_layout_passes=False),
)
def _sc_main(tp_hbm, rays_hbm, b_hbm, out_hbm,
             rays_v, b_v, idx_v, g_v, out_v, sem):
    wid = lax.axis_index("s") * 2 + lax.axis_index("c")
    base = pl.multiple_of(wid * 128, 128)
    pltpu.sync_copy(rays_hbm.at[pl.ds(pl.multiple_of(wid * 1024, 1024), 1024)],
                    rays_v)
    pltpu.sync_copy(b_hbm, b_v)
    bv = b_v[...]
    lane = jnp.arange(16, dtype=jnp.int32)

    for grp in range(8):
        rvec = (grp * 16 + lane) * 8
        pa = [plsc.load_gather(rays_v, [rvec + p]) for p in range(6)]
        ax, ay, az, dx, dy, dz = pa

        def body(it, m):
            tf = it.astype(jnp.float32) * jnp.float32(1.0 / 199.0)
            x0 = jnp.clip(ax + dx * tf, 0.0, 1.0)
            x1 = jnp.clip(ay + dy * tf, 0.0, 1.0)
            x2 = jnp.clip(az + dz * tf, 0.0, 1.0)
            wlist = []
            for l in range(_NUM_LEVELS):
                r1 = jnp.float32(_RES[l] - 1)
                p0 = x0 * r1
                p1 = x1 * r1
                p2 = x2 * r1
                i0 = p0.astype(jnp.int32)
                i1 = p1.astype(jnp.int32)
                i2 = p2.astype(jnp.int32)
                f0 = p0 - i0.astype(jnp.float32)
                f1 = p1 - i1.astype(jnp.float32)
                f2 = p2 - i2.astype(jnp.float32)
                g0 = 1.0 - f0
                g1 = 1.0 - f1
                g2 = 1.0 - f2
                yz = (g1 * g2, f1 * g2, g1 * f2, f1 * f2)
                if _DENSE[l]:
                    s = _RES[l] + 1
                    bidx = i0 + i1 * s + i2 * (s * s) + _OFF[l]
                else:
                    y0 = i1 * _P1
                    y1 = y0 + _P1
                    z0 = i2 * _P2
                    z1 = z0 + _P2
                    msk = np.int32(_TS[l] - 1)
                for c8 in range(8):
                    ox, oy, oz = c8 & 1, (c8 >> 1) & 1, (c8 >> 2) & 1
                    w = (f0 if ox else g0) * yz[oy + 2 * oz]
                    wlist.append(w)
                    if _DENSE[l]:
                        s = _RES[l] + 1
                        idx = bidx + (ox + oy * s + oz * (s * s))
                    else:
                        tx = (i0 + 1) if ox else i0
                        ty = y1 if oy else y0
                        tz = z1 if oz else z0
                        idx = ((tx ^ ty ^ tz) & msk) + _OFF[l]
                    idx_v[l, pl.ds(c8 * 16, 16)] = idx
            cps = [pltpu.async_copy(tp_hbm.at[idx_v.at[l]], g_v.at[l], sem)
                   for l in range(_NUM_LEVELS)]
            for cp in cps:
                cp.wait()
            z = jnp.zeros((16,), jnp.float32)
            k = 0
            for l in range(_NUM_LEVELS):
                for c8 in range(8):
                    z = z + wlist[k] * g_v[l, pl.ds(c8 * 16, 16)]
                    k += 1
            return jnp.maximum(m, z)

        m = lax.fori_loop(0, _N_POINTS, body,
                          jnp.full((16,), -3e38, jnp.float32))
        out_v[pl.ds(grp * 16, 16)] = 1.0 / (1.0 + jnp.exp(-(m + bv)))

    pltpu.sync_copy(out_v, out_hbm.at[pl.ds(base, 128)])


def kernel(x, table0, table1, table2, table3, W, b):
    tabs = (table0, table1, table2, table3)
    projs = [_project_level(l, tabs[l], W) for l in range(_NUM_LEVELS)]
    projs.append(jnp.zeros((_TPTOT_PAD - _TPTOT,), jnp.float32))
    tp = jnp.concatenate(projs)
    rays = _rays_tc(x).reshape(-1)
    bvec = jnp.broadcast_to(b.astype(jnp.float32), (16,))
    out = _sc_main(tp, rays, bvec)
    return out.reshape(_N_RAYS, 1)


# speedup vs baseline: 121.3771x; 4.0901x over previous
import functools

import numpy as np
import jax
import jax.numpy as jnp
from jax import lax
from jax.experimental import pallas as pl
from jax.experimental.pallas import tpu as pltpu
from jax.experimental.pallas import tpu_sc as plsc

_N_RAYS = 4096
_N_POINTS = 200
_NUM_LEVELS = 4
_RES = (32, 64, 128, 256)
_TS = (35937, 274625, 524288, 524288)
_P1 = np.int32(np.uint32(2654435761).view(np.int32))
_P2 = np.int32(805459861)

_T0_PAD = 35940
_T1_PAD = 274688
_T23_WORDS = 524288
_L3_WOFF = 262144
_CH1 = _T1_PAD // 16
_CH23 = _T23_WORDS // 16

_SEL = (np.arange(128)[:, None] // 8 == np.arange(16)[None, :]).astype(np.float32)


def _proj_body(t_ref, s_ref, o_ref):
    o_ref[...] = jnp.dot(t_ref[...], s_ref[...],
                         preferred_element_type=jnp.float32)


def _project_level(l, table, W):
    T = _TS[l]
    Tp = ((T + 15) // 16) * 16
    R = Tp // 16
    tf = jnp.pad(table, ((0, Tp - T), (0, 0))).reshape(R, 128)
    wl = W[8 * l:8 * l + 8, 0]
    S = jnp.asarray(_SEL) * jnp.tile(wl, 16)[:, None]
    BR = 1024
    grid = (R + BR - 1) // BR
    out = pl.pallas_call(
        _proj_body,
        grid=(grid,),
        in_specs=[pl.BlockSpec((BR, 128), lambda i: (i, 0)),
                  pl.BlockSpec((128, 16), lambda i: (0, 0))],
        out_specs=pl.BlockSpec((BR, 16), lambda i: (i, 0)),
        out_shape=jax.ShapeDtypeStruct((R, 16), jnp.float32),
    )(tf, S)
    return out.reshape(-1)[:T]


def _rays_body(x_ref, o_ref):
    xb = x_ref[...]
    th1, ph1, th2, ph2 = (xb[:, i:i + 1] for i in range(4))
    s1, c1 = jnp.sin(th1), jnp.cos(th1)
    s2, c2 = jnp.sin(th2), jnp.cos(th2)
    p1 = (s1 * jnp.cos(ph1), s1 * jnp.sin(ph1), c1)
    p2 = (s2 * jnp.cos(ph2), s2 * jnp.sin(ph2), c2)
    cols = [(p1[k] + 1.0) * 0.5 for k in range(3)]
    cols += [(p2[k] - p1[k]) * 0.5 for k in range(3)]
    z = jnp.zeros_like(th1)
    o_ref[...] = jnp.concatenate(cols + [z, z], axis=1)


def _rays_tc(x):
    return pl.pallas_call(
        _rays_body,
        out_shape=jax.ShapeDtypeStruct((_N_RAYS, 8), jnp.float32),
    )(x)


_mesh = plsc.VectorSubcoreMesh(core_axis_name="c", subcore_axis_name="s",
                               num_cores=2, num_subcores=16)


@functools.partial(
    pl.kernel,
    out_type=jax.ShapeDtypeStruct((_N_RAYS,), jnp.float32),
    mesh=_mesh,
    scratch_types=[
        pltpu.VMEM((_T0_PAD,), jnp.float32),
        pltpu.VMEM((1024,), jnp.float32),
        pltpu.VMEM((16,), jnp.float32),
        pltpu.VMEM((2, 128), jnp.int32),
        pltpu.VMEM((2, 2, 128), jnp.int32),
        pltpu.VMEM((2, 2, 128), jnp.int32),
        pltpu.VMEM((2, 128), jnp.float32),
        pltpu.VMEM((2, 2, 128), jnp.int32),
        pltpu.VMEM((128,), jnp.float32),
        pltpu.VMEM((_CH1 // 2,), jnp.float32),
        pltpu.VMEM((_CH23 // 4,), jnp.int32),
        pltpu.VMEM_SHARED((_T1_PAD,), jnp.float32),
        pltpu.VMEM_SHARED((_T23_WORDS,), jnp.int32),
        pltpu.SemaphoreType.DMA,
        pltpu.SemaphoreType.DMA,
    ],
    compiler_params=pltpu.CompilerParams(needs_layout_passes=False),
)
def _sc_main(tp0_hbm, tp1_hbm, tp23_hbm, rays_hbm, b_hbm, out_hbm,
             tp0_v, rays_v, b_v, idx1_v, idx23_v, psh_v, g1_v, g23_v,
             out_v, st1_v, st23_v, spm1, spm23, sem0, sem1):
    sid = lax.axis_index("s")
    wid = sid * 2 + lax.axis_index("c")
    base = pl.multiple_of(wid * 128, 128)
    pltpu.sync_copy(tp0_hbm, tp0_v)
    for r in range(2):
        o1 = pl.multiple_of(sid * _CH1 + r * (_CH1 // 2), 8)
        pltpu.sync_copy(tp1_hbm.at[pl.ds(o1, _CH1 // 2)], st1_v)
        pltpu.sync_copy(st1_v, spm1.at[pl.ds(o1, _CH1 // 2)])
    for r in range(4):
        o23 = pl.multiple_of(sid * _CH23 + r * (_CH23 // 4), 8)
        pltpu.sync_copy(tp23_hbm.at[pl.ds(o23, _CH23 // 4)], st23_v)
        pltpu.sync_copy(st23_v, spm23.at[pl.ds(o23, _CH23 // 4)])
    pltpu.sync_copy(rays_hbm.at[pl.ds(pl.multiple_of(wid * 1024, 1024), 1024)],
                    rays_v)
    pltpu.sync_copy(b_hbm, b_v)
    bv = b_v[...]
    lane = jnp.arange(16, dtype=jnp.int32)
    plsc.subcore_barrier()

    sems = (sem0, sem1)

    def fire(buf):
        sem = sems[buf]
        pltpu.async_copy(spm1.at[idx1_v.at[buf]], g1_v.at[buf], sem)
        pltpu.async_copy(spm23.at[idx23_v.at[buf, 0]], g23_v.at[buf, 0], sem)
        pltpu.async_copy(spm23.at[idx23_v.at[buf, 1]], g23_v.at[buf, 1], sem)

    def drain(buf):
        sem = sems[buf]
        pltpu.make_async_copy(spm1.at[idx1_v.at[buf]], g1_v.at[buf],
                              sem).wait()
        pltpu.make_async_copy(spm23.at[idx23_v.at[buf, 0]], g23_v.at[buf, 0],
                              sem).wait()
        pltpu.make_async_copy(spm23.at[idx23_v.at[buf, 1]], g23_v.at[buf, 1],
                              sem).wait()

    for grp in range(8):
        rvec = (grp * 16 + lane) * 8
        pa = [plsc.load_gather(rays_v, [rvec + p]) for p in range(6)]
        ax, ay, az, dx, dy, dz = pa

        def prep(it, buf):
            tf = it.astype(jnp.float32) * jnp.float32(1.0 / 199.0)
            x0 = jnp.clip(ax + dx * tf, 0.0, 1.0)
            x1 = jnp.clip(ay + dy * tf, 0.0, 1.0)
            x2 = jnp.clip(az + dz * tf, 0.0, 1.0)
            carry_w = []
            z = jnp.zeros((16,), jnp.float32)
            for l in range(_NUM_LEVELS):
                r1 = jnp.float32(_RES[l] - 1)
                p0 = x0 * r1
                p1 = x1 * r1
                p2 = x2 * r1
                i0 = p0.astype(jnp.int32)
                i1 = p1.astype(jnp.int32)
                i2 = p2.astype(jnp.int32)
                f0 = p0 - i0.astype(jnp.float32)
                f1 = p1 - i1.astype(jnp.float32)
                f2 = p2 - i2.astype(jnp.float32)
                g0 = 1.0 - f0
                g1 = 1.0 - f1
                g2 = 1.0 - f2
                yz = (g1 * g2, f1 * g2, g1 * f2, f1 * f2)
                if l < 2:
                    s = _RES[l] + 1
                    bidx = i0 + i1 * s + i2 * (s * s)
                else:
                    y0 = i1 * _P1
                    y1 = y0 + _P1
                    z0 = i2 * _P2
                    z1 = z0 + _P2
                    msk = np.int32(_TS[l] - 1)
                for c8 in range(8):
                    ox, oy, oz = c8 & 1, (c8 >> 1) & 1, (c8 >> 2) & 1
                    w = (f0 if ox else g0) * yz[oy + 2 * oz]
                    if l < 2:
                        s = _RES[l] + 1
                        idx = bidx + (ox + oy * s + oz * (s * s))
                        if l == 0:
                            z = z + w * plsc.load_gather(tp0_v, [idx])
                        else:
                            idx1_v[buf, pl.ds(c8 * 16, 16)] = idx
                            carry_w.append(w)
                    else:
                        tx = (i0 + 1) if ox else i0
                        ty = y1 if oy else y0
                        tz = z1 if oz else z0
                        h = (tx ^ ty ^ tz) & msk
                        word = lax.shift_right_logical(h, 1)
                        if l == 3:
                            word = word + _L3_WOFF
                        sh = lax.shift_left((h & 1) ^ 1, 4)
                        idx23_v[buf, l - 2, pl.ds(c8 * 16, 16)] = word
                        psh_v[buf, l - 2, pl.ds(c8 * 16, 16)] = sh
                        carry_w.append(w)
            return (z,) + tuple(carry_w)

        def consume(buf, wl, m):
            drain(buf)
            z = wl[0]
            for c8 in range(8):
                z = z + wl[1 + c8] * g1_v[buf, pl.ds(c8 * 16, 16)]
            for l2 in range(2):
                for c8 in range(8):
                    w32 = g23_v[buf, l2, pl.ds(c8 * 16, 16)]
                    sh = psh_v[buf, l2, pl.ds(c8 * 16, 16)]
                    bits = lax.shift_left(w32, sh) & np.int32(-65536)
                    val = plsc.bitcast(bits, jnp.float32)
                    z = z + wl[9 + l2 * 8 + c8] * val
            return jnp.maximum(m, z)

        w0 = prep(jnp.int32(0), 0)
        fire(0)

        def body(j, carry):
            m, wa = carry
            t0 = j * 2
            wb = prep(t0 + 1, 1)
            fire(1)
            m = consume(0, wa, m)
            wa2 = prep(t0 + 2, 0)
            fire(0)
            m = consume(1, wb, m)
            return (m, tuple(wa2))

        m, _wlast = lax.fori_loop(
            0, _N_POINTS // 2,
            body, (jnp.full((16,), -3e38, jnp.float32), tuple(w0)))
        drain(0)
        out_v[pl.ds(grp * 16, 16)] = 1.0 / (1.0 + jnp.exp(-(m + bv)))

    pltpu.sync_copy(out_v, out_hbm.at[pl.ds(base, 128)])


def kernel(x, table0, table1, table2, table3, W, b):
    tabs = (table0, table1, table2, table3)
    projs = [_project_level(l, tabs[l], W) for l in range(_NUM_LEVELS)]
    tp0 = jnp.concatenate([projs[0],
                           jnp.zeros((_T0_PAD - _TS[0],), jnp.float32)])
    tp1 = jnp.concatenate([projs[1],
                           jnp.zeros((_T1_PAD - _TS[1],), jnp.float32)])
    packed = [lax.bitcast_convert_type(
        projs[l].astype(jnp.bfloat16).reshape(-1, 2), jnp.int32)
        for l in (2, 3)]
    tp23 = jnp.concatenate(packed)
    rays = _rays_tc(x).reshape(-1)
    bvec = jnp.broadcast_to(b.astype(jnp.float32), (16,))
    out = _sc_main(tp0, tp1, tp23, rays, bvec)
    return out.reshape(_N_RAYS, 1)
